# baseline (speedup 1.0000x reference)
import functools

import jax
import jax.numpy as jnp
from jax import lax
from jax.experimental import pallas as pl
from jax.experimental.pallas import tpu as pltpu
from jax.experimental.pallas import tpu_sc as plsc

N_POS = 50
ALPHA = 0.3
NP = 128
KH = 8
KA = 8
NC = 2
NS = 16
NW = NC * NS
IC = 128


def _split(x):
    hi = x.astype(jnp.bfloat16)
    lo = (x - hi.astype(jnp.float32)).astype(jnp.bfloat16)
    return hi, lo


def _dot_oh(onehot_bf16, dense_f32, dnums, onehot_lhs=True):
    hi, lo = _split(dense_f32)
    d = functools.partial(jax.lax.dot_general, dimension_numbers=dnums,
                          preferred_element_type=jnp.float32)
    if onehot_lhs:
        return d(onehot_bf16, hi) + d(onehot_bf16, lo)
    return d(hi, onehot_bf16) + d(lo, onehot_bf16)


def _rowseg_body(*refs):
    adds_refs, a_refs, out_ref = refs[:KH], refs[KH:2 * KH], refs[-1]
    for k, (adds_r, a_ref) in enumerate(zip(adds_refs, a_refs)):
        adds_row = adds_r[0]
        qs = jax.lax.broadcasted_iota(jnp.int32, (NP, adds_row.shape[1]), 0)
        et = (qs == adds_row).astype(jnp.bfloat16)
        out_ref[k] = _dot_oh(et, a_ref[0], (((1,), (1,)), ((), ())),
                             onehot_lhs=False)


def _sc_hist_body(m_hbm, heads_hbm, zero_hbm, gparts_hbm,
                  idx_v, rows_v, gsh, sems):
    cid = lax.axis_index("c")
    sid = lax.axis_index("s")
    wid = sid * NC + cid
    nchunks = heads_hbm.shape[1]
    rpw = nchunks * IC

    @pl.when(sid == 0)
    def _():
        pltpu.sync_copy(zero_hbm, gsh)

    plsc.subcore_barrier()
    pltpu.sync_copy(heads_hbm.at[wid], idx_v)

    def _m_copy(c, slot):
        return pltpu.make_async_copy(
            m_hbm.at[pl.ds(wid * rpw + c * IC, IC)], rows_v.at[slot],
            sems.at[slot])

    _m_copy(0, 0).start()
    for c in range(nchunks):
        slot = c % 2
        if c + 1 < nchunks:
            _m_copy(c + 1, 1 - slot).start()
        _m_copy(c, slot).wait()
        pltpu.sync_copy(rows_v.at[slot], gsh.at[idx_v.at[c]], add=True)
    plsc.subcore_barrier()

    @pl.when(sid == 0)
    def _():
        pltpu.sync_copy(gsh, gparts_hbm.at[cid])


def _apply_body(*refs):
    pos_ref, g_refs = refs[0], refs[1:3]
    s_refs, out_ref = refs[3:3 + KA], refs[-1]
    gtot = g_refs[0][0] + g_refs[0][1]
    for gr in g_refs[1:]:
        gtot = gtot + gr[0] + gr[1]
    gs = jax.nn.sigmoid(gtot)
    for k, s_ref in enumerate(s_refs):
        pos_row = pos_ref[k]
        qs = jax.lax.broadcasted_iota(jnp.int32, (NP, pos_row.shape[1]), 0)
        pt = (qs == pos_row).astype(jnp.bfloat16)
        u = _dot_oh(pt, gs, (((0,), (0,)), ((), ())))
        add = _dot_oh(pt, u, (((1,), (0,)), ((), ())), onehot_lhs=False)
        out_ref[k] = s_ref[0] + ALPHA * add


@jax.jit
def kernel(words, feats, adds, pos, s_arc, a_arc):
    del words, feats
    B, S = adds.shape
    adds3 = adds.reshape(B, 1, S)
    pos3 = pos.reshape(B, 1, S)

    zero = jnp.zeros((NP, NP), jnp.float32)

    def _make_sc_hist(nchunks):
        return pl.kernel(
            _sc_hist_body,
            out_type=jax.ShapeDtypeStruct((NC, NP, NP), jnp.float32),
            mesh=plsc.VectorSubcoreMesh(core_axis_name="c",
                                        subcore_axis_name="s"),
            scratch_types=[
                pltpu.VMEM((nchunks, IC), jnp.int32),
                pltpu.VMEM((2, IC, NP), jnp.float32),
                pltpu.VMEM_SHARED((NP, NP), jnp.float32),
                pltpu.SemaphoreType.DMA((2,)),
            ],
        )

    rounds = [(0, 40), (40, 24)]
    gparts = []
    for b0, bn in rounds:
        hmaps = [(lambda b, k=k, b0=b0: (b0 + KH * b + k, 0, 0))
                 for k in range(KH)]
        m = pl.pallas_call(
            _rowseg_body,
            grid=(bn // KH,),
            in_specs=[pl.BlockSpec((1, 1, S), m_) for m_ in hmaps]
                     + [pl.BlockSpec((1, S, S), m_) for m_ in hmaps],
            out_specs=pl.BlockSpec((KH, S, NP), lambda b: (b, 0, 0)),
            out_shape=jax.ShapeDtypeStruct((bn, S, NP), jnp.float32),
        )(*([adds3] * KH), *([a_arc] * KH))
        rpw = (bn * S) // NW
        heads3 = adds[b0:b0 + bn].reshape(NW, rpw // IC, IC)
        gparts.append(_make_sc_hist(rpw // IC)(
            m.reshape(bn * S, NP), heads3, zero))

    amaps = [(lambda b, k=k: (KA * b + k, 0, 0)) for k in range(KA)]
    out = pl.pallas_call(
        _apply_body,
        grid=(B // KA,),
        in_specs=[
            pl.BlockSpec((KA, 1, S), lambda b: (b, 0, 0)),
        ] + [pl.BlockSpec((NC, NP, NP), lambda b: (0, 0, 0))
             for _ in range(2)]
          + [pl.BlockSpec((1, S, S), m_) for m_ in amaps],
        out_specs=pl.BlockSpec((KA, S, S), lambda b: (b, 0, 0)),
        out_shape=jax.ShapeDtypeStruct((B, S, S), jnp.float32),
    )(pos3, *gparts, *([s_arc] * KA))
    return out


# speedup vs baseline: 1.0120x; 1.0120x over previous
import functools

import jax
import jax.numpy as jnp
from jax import lax
from jax.experimental import pallas as pl
from jax.experimental.pallas import tpu as pltpu
from jax.experimental.pallas import tpu_sc as plsc

N_POS = 50
ALPHA = 0.3
NP = 128
KH = 8
KA = 8
NC = 2
NS = 16
NW = NC * NS
IC = 128


def _split(x):
    hi = x.astype(jnp.bfloat16)
    lo = (x - hi.astype(jnp.float32)).astype(jnp.bfloat16)
    return hi, lo


def _dot_oh(onehot_bf16, dense_f32, dnums, onehot_lhs=True):
    hi, lo = _split(dense_f32)
    d = functools.partial(jax.lax.dot_general, dimension_numbers=dnums,
                          preferred_element_type=jnp.float32)
    if onehot_lhs:
        return d(onehot_bf16, hi) + d(onehot_bf16, lo)
    return d(hi, onehot_bf16) + d(lo, onehot_bf16)


def _rowseg_body(*refs):
    adds_refs, a_refs, out_ref = refs[:KH], refs[KH:2 * KH], refs[-1]
    for k, (adds_r, a_ref) in enumerate(zip(adds_refs, a_refs)):
        adds_row = adds_r[0]
        qs = jax.lax.broadcasted_iota(jnp.int32, (NP, adds_row.shape[1]), 0)
        et = (qs == adds_row).astype(jnp.bfloat16)
        out_ref[k] = _dot_oh(et, a_ref[0], (((1,), (1,)), ((), ())),
                             onehot_lhs=False)


def _sc_hist_body(m_hbm, heads_hbm, zero_hbm, gparts_hbm,
                  idx_v, rows_v, gsh, sems):
    cid = lax.axis_index("c")
    sid = lax.axis_index("s")
    wid = sid * NC + cid
    nchunks = heads_hbm.shape[1]
    rpw = nchunks * IC

    @pl.when(sid == 0)
    def _():
        pltpu.sync_copy(zero_hbm, gsh)

    plsc.subcore_barrier()
    pltpu.sync_copy(heads_hbm.at[wid], idx_v)

    def _m_copy(c, slot):
        return pltpu.make_async_copy(
            m_hbm.at[pl.ds(wid * rpw + c * IC, IC)], rows_v.at[slot],
            sems.at[slot])

    _m_copy(0, 0).start()
    for c in range(nchunks):
        slot = c % 2
        if c + 1 < nchunks:
            _m_copy(c + 1, 1 - slot).start()
        _m_copy(c, slot).wait()
        pltpu.sync_copy(rows_v.at[slot], gsh.at[idx_v.at[c]], add=True)
    plsc.subcore_barrier()

    @pl.when(sid == 0)
    def _():
        pltpu.sync_copy(gsh, gparts_hbm.at[cid])


def _apply_body(*refs):
    pos_ref, g_refs = refs[0], refs[1:3]
    s_refs, out_ref = refs[3:3 + KA], refs[-1]
    gtot = g_refs[0][0] + g_refs[0][1]
    for gr in g_refs[1:]:
        gtot = gtot + gr[0] + gr[1]
    gs = jax.nn.sigmoid(gtot)
    for k, s_ref in enumerate(s_refs):
        pos_row = pos_ref[k]
        qs = jax.lax.broadcasted_iota(jnp.int32, (NP, pos_row.shape[1]), 0)
        pt = (qs == pos_row).astype(jnp.bfloat16)
        u = _dot_oh(pt, gs, (((0,), (0,)), ((), ())))
        add = _dot_oh(pt, u, (((1,), (0,)), ((), ())), onehot_lhs=False)
        out_ref[k] = s_ref[0] + ALPHA * add


@jax.jit
def kernel(words, feats, adds, pos, s_arc, a_arc):
    del words, feats
    B, S = adds.shape
    adds3 = adds.reshape(B, 1, S)
    pos3 = pos.reshape(B, 1, S)

    zero = jnp.zeros((NP, NP), jnp.float32)

    def _make_sc_hist(nchunks):
        return pl.kernel(
            _sc_hist_body,
            out_type=jax.ShapeDtypeStruct((NC, NP, NP), jnp.float32),
            mesh=plsc.VectorSubcoreMesh(core_axis_name="c",
                                        subcore_axis_name="s"),
            scratch_types=[
                pltpu.VMEM((nchunks, IC), jnp.int32),
                pltpu.VMEM((2, IC, NP), jnp.float32),
                pltpu.VMEM_SHARED((NP, NP), jnp.float32),
                pltpu.SemaphoreType.DMA((2,)),
            ],
        )

    rounds = [(0, 32), (32, 32)]
    gparts = []
    for b0, bn in rounds:
        hmaps = [(lambda b, k=k, b0=b0: (b0 + KH * b + k, 0, 0))
                 for k in range(KH)]
        m = pl.pallas_call(
            _rowseg_body,
            grid=(bn // KH,),
            in_specs=[pl.BlockSpec((1, 1, S), m_) for m_ in hmaps]
                     + [pl.BlockSpec((1, S, S), m_) for m_ in hmaps],
            out_specs=pl.BlockSpec((KH, S, NP), lambda b: (b, 0, 0)),
            out_shape=jax.ShapeDtypeStruct((bn, S, NP), jnp.float32),
        )(*([adds3] * KH), *([a_arc] * KH))
        rpw = (bn * S) // NW
        heads3 = adds[b0:b0 + bn].reshape(NW, rpw // IC, IC)
        gparts.append(_make_sc_hist(rpw // IC)(
            m.reshape(bn * S, NP), heads3, zero))

    amaps = [(lambda b, k=k: (KA * b + k, 0, 0)) for k in range(KA)]
    out = pl.pallas_call(
        _apply_body,
        grid=(B // KA,),
        in_specs=[
            pl.BlockSpec((KA, 1, S), lambda b: (b, 0, 0)),
        ] + [pl.BlockSpec((NC, NP, NP), lambda b: (0, 0, 0))
             for _ in range(2)]
          + [pl.BlockSpec((1, S, S), m_) for m_ in amaps],
        out_specs=pl.BlockSpec((KA, S, S), lambda b: (b, 0, 0)),
        out_shape=jax.ShapeDtypeStruct((B, S, S), jnp.float32),
    )(pos3, *gparts, *([s_arc] * KA))
    return out
